# Initial kernel scaffold; baseline (speedup 1.0000x reference)
import jax
import jax.numpy as jnp
from jax.experimental import pallas as pl


def kernel(x, W_embed, b_embed, W_gate, Wi, bi, Wo, bo, W_proj, b_proj):
    raise NotImplementedError("write your pallas kernel here")


# speedup vs baseline: 21.6846x; 21.6846x over previous
import jax
import jax.numpy as jnp
from jax.experimental import pallas as pl
from jax.experimental.pallas import tpu as pltpu

T = 32768
D_IN = 4
D_HID = 16
D_FF = 32
E = 8
TT = 4096


def _moe_block(x_ref, We_ref, be_ref, Wg_ref, Wif_ref, bif_ref, Wof_ref,
               bo_ref, Wp_ref, bp_ref, out_ref):
    x = x_ref[...]
    h = jnp.dot(x, We_ref[...], preferred_element_type=jnp.float32) + be_ref[...]
    logits = jnp.dot(h, Wg_ref[...], preferred_element_type=jnp.float32)
    m = jnp.max(logits, axis=-1, keepdims=True)
    el = jnp.exp(logits - m)
    gate = 1.0 / jnp.sum(el, axis=-1, keepdims=True)
    eids = jax.lax.broadcasted_iota(jnp.int32, logits.shape, 1)
    idx = jnp.min(jnp.where(logits >= m, eids, E), axis=-1, keepdims=True)
    mid = jax.nn.gelu(
        jnp.dot(h, Wif_ref[...], preferred_element_type=jnp.float32) + bif_ref[...])
    col_e = jax.lax.broadcasted_iota(jnp.int32, mid.shape, 1) // D_FF
    midm = jnp.where(col_e == idx, mid, 0.0)
    onehot = (eids == idx).astype(jnp.float32)
    moe = (jnp.dot(midm, Wof_ref[...], preferred_element_type=jnp.float32)
           + jnp.dot(onehot, bo_ref[...], preferred_element_type=jnp.float32))
    moe = moe * gate
    out_ref[...] = (jnp.dot(moe, Wp_ref[...], preferred_element_type=jnp.float32)
                    + bp_ref[...])


def kernel(x, W_embed, b_embed, W_gate, Wi, bi, Wo, bo, W_proj, b_proj):
    Wi_flat = Wi.transpose(1, 0, 2).reshape(D_HID, E * D_FF)
    bi_flat = bi.reshape(1, E * D_FF)
    Wo_flat = Wo.reshape(E * D_FF, D_HID)
    full = lambda a: pl.BlockSpec(a.shape, lambda i: (0,) * a.ndim)
    args = (W_embed, b_embed.reshape(1, D_HID), W_gate, Wi_flat, bi_flat,
            Wo_flat, bo, W_proj, b_proj.reshape(1, D_IN))
    return pl.pallas_call(
        _moe_block,
        grid=(T // TT,),
        in_specs=[pl.BlockSpec((TT, D_IN), lambda i: (i, 0))]
                 + [full(a) for a in args],
        out_specs=pl.BlockSpec((TT, D_IN), lambda i: (i, 0)),
        out_shape=jax.ShapeDtypeStruct((T, D_IN), jnp.float32),
        compiler_params=pltpu.CompilerParams(
            dimension_semantics=("parallel",)),
    )(x, *args)


# speedup vs baseline: 22.9018x; 1.0561x over previous
import jax
import jax.numpy as jnp
import numpy as np
from jax import lax
from jax.experimental import pallas as pl
from jax.experimental.pallas import tpu as pltpu

T = 32768
D_IN = 4
D_HID = 16
D_FF = 32
E = 8
TT = 4096

_EYE_E = np.eye(E, dtype=np.float32)
_TILE16_T = np.tile(np.eye(D_HID, dtype=np.float32), (E, 1))
_TILE32_T = np.tile(np.eye(D_FF, dtype=np.float32), (E, 1))
_S128_T = np.repeat(_EYE_E, D_HID, axis=0)
_S256_T = np.repeat(_EYE_E, D_FF, axis=0)
_PREFIX_T = np.tril(np.ones((E, E), dtype=np.float32), -1)


def _moe_block(x_ref, WeT_ref, beT_ref, WgT_ref, WiRowsT_ref, s128bi_ref,
               WoRowsT_ref, boT_ref, WpT_ref, bpT_ref, tile16_ref, tile32_ref,
               s256_ref, prefix_ref, out_ref):
    f32 = jnp.float32
    dot = lambda a, b: jnp.dot(a, b, preferred_element_type=f32)
    x = x_ref[...]
    h = lax.dot_general(WeT_ref[...], x, (((1,), (1,)), ((), ())),
                        preferred_element_type=f32) + beT_ref[...]
    logits = dot(WgT_ref[...], h)
    m = jnp.max(logits, axis=0, keepdims=True)
    el = jnp.exp(logits - m)
    gate = 1.0 / jnp.sum(el, axis=0, keepdims=True)
    is_max = (logits >= m).astype(f32)
    prior = dot(prefix_ref[...], is_max)
    onehot = jnp.where(prior == 0.0, is_max, 0.0)
    sel1 = dot(s128bi_ref[...], onehot)
    h_exp = dot(tile16_ref[...], h) * sel1[:E * D_HID]
    pre = dot(WiRowsT_ref[...], h_exp) + sel1[E * D_HID:]
    mid = jax.nn.gelu(pre)
    m_exp = dot(tile32_ref[...], mid) * dot(s256_ref[...], onehot)
    moe = (dot(WoRowsT_ref[...], m_exp) + dot(boT_ref[...], onehot)) * gate
    out_t = dot(WpT_ref[...], moe) + bpT_ref[...]
    out_ref[...] = out_t.T


def kernel(x, W_embed, b_embed, W_gate, Wi, bi, Wo, bo, W_proj, b_proj):
    WiRowsT = Wi.reshape(E * D_HID, D_FF).T
    WoRowsT = Wo.reshape(E * D_FF, D_HID).T
    full = lambda a: pl.BlockSpec(a.shape, lambda i: (0,) * a.ndim)
    s128bi = jnp.concatenate([jnp.asarray(_S128_T), bi.T], axis=0)
    args = (W_embed.T, b_embed.reshape(D_HID, 1), W_gate.T, WiRowsT, s128bi,
            WoRowsT, bo.T, W_proj.T, b_proj.reshape(D_IN, 1),
            jnp.asarray(_TILE16_T), jnp.asarray(_TILE32_T),
            jnp.asarray(_S256_T), jnp.asarray(_PREFIX_T))
    return pl.pallas_call(
        _moe_block,
        grid=(T // TT,),
        in_specs=[pl.BlockSpec((TT, D_IN), lambda i: (i, 0))]
                 + [full(a) for a in args],
        out_specs=pl.BlockSpec((TT, D_IN), lambda i: (i, 0)),
        out_shape=jax.ShapeDtypeStruct((T, D_IN), jnp.float32),
        compiler_params=pltpu.CompilerParams(
            dimension_semantics=("parallel",)),
    )(x, *args)


# speedup vs baseline: 24.3603x; 1.0637x over previous
import jax
import jax.numpy as jnp
import numpy as np
from jax import lax
from jax.experimental import pallas as pl
from jax.experimental.pallas import tpu as pltpu

T = 32768
D_IN = 4
D_HID = 16
D_FF = 32
E = 8
TT = 8192

_EYE_E = np.eye(E, dtype=np.float32)
_TILE16_T = np.tile(np.eye(D_HID, dtype=np.float32), (E, 1))
_TILE32_T = np.tile(np.eye(D_FF, dtype=np.float32), (E, 1))
_S128_T = np.repeat(_EYE_E, D_HID, axis=0)
_S256_T = np.repeat(_EYE_E, D_FF, axis=0)
_PREFIX_T = np.tril(np.ones((E, E), dtype=np.float32), -1)

_NN = (((1,), (0,)), ((), ()))
_TN = (((0,), (0,)), ((), ()))
_TX = (((0,), (1,)), ((), ()))


def _moe_block(x_ref, We_ref, be_ref, Wg_ref, Wi_ref, bi_ref, Wo_ref, bo_ref,
               Wp_ref, bp_ref, tile16_ref, tile32_ref, s128_ref, s256_ref,
               prefix_ref, out_ref):
    f32 = jnp.float32
    dot = lambda a, b, dn=_NN: lax.dot_general(a, b, dn,
                                               preferred_element_type=f32)
    x = x_ref[...]
    h = dot(We_ref[...], x, _TX) + be_ref[...]
    logits = dot(Wg_ref[...], h, _TN)
    m = jnp.max(logits, axis=0, keepdims=True)
    el = jnp.exp(logits - m)
    gate = 1.0 / jnp.sum(el, axis=0, keepdims=True)
    is_max = (logits >= m).astype(f32)
    prior = dot(prefix_ref[...], is_max)
    onehot = jnp.where(prior == 0.0, is_max, 0.0)
    h_exp = dot(tile16_ref[...], h) * dot(s128_ref[...], onehot)
    wi_rows = Wi_ref[...].reshape(E * D_HID, D_FF)
    pre = dot(wi_rows, h_exp, _TN) + dot(bi_ref[...], onehot, _TN)
    mid = jax.nn.gelu(pre)
    m_exp = dot(tile32_ref[...], mid) * dot(s256_ref[...], onehot)
    wo_rows = Wo_ref[...].reshape(E * D_FF, D_HID)
    moe = (dot(wo_rows, m_exp, _TN) + dot(bo_ref[...], onehot, _TN)) * gate
    out_t = dot(Wp_ref[...], moe, _TN) + bp_ref[...]
    out_ref[...] = out_t.T


def kernel(x, W_embed, b_embed, W_gate, Wi, bi, Wo, bo, W_proj, b_proj):
    full = lambda a: pl.BlockSpec(a.shape, lambda i: (0,) * a.ndim)
    args = (W_embed, b_embed.reshape(D_HID, 1), W_gate, Wi, bi, Wo, bo,
            W_proj, b_proj.reshape(D_IN, 1),
            jnp.asarray(_TILE16_T), jnp.asarray(_TILE32_T),
            jnp.asarray(_S128_T), jnp.asarray(_S256_T), jnp.asarray(_PREFIX_T))
    return pl.pallas_call(
        _moe_block,
        grid=(T // TT,),
        in_specs=[pl.BlockSpec((TT, D_IN), lambda i: (i, 0))]
                 + [full(a) for a in args],
        out_specs=pl.BlockSpec((TT, D_IN), lambda i: (i, 0)),
        out_shape=jax.ShapeDtypeStruct((T, D_IN), jnp.float32),
        compiler_params=pltpu.CompilerParams(
            dimension_semantics=("parallel",)),
    )(x, *args)


# speedup vs baseline: 28.1199x; 1.1543x over previous
import jax
import jax.numpy as jnp
import numpy as np
from jax import lax
from jax.experimental import pallas as pl
from jax.experimental.pallas import tpu as pltpu

T = 32768
D_IN = 4
D_HID = 16
D_FF = 32
E = 8
TT = 8192

_EYE_E = np.eye(E, dtype=np.float32)
_TILE16_T = np.tile(np.eye(D_HID, dtype=np.float32), (E, 1))
_GSUM = np.tile(np.eye(D_HID, dtype=np.float32), (1, E))
_S128_T = np.repeat(_EYE_E, D_HID, axis=0)
_PREFIX_T = np.tril(np.ones((E, E), dtype=np.float32), -1)

_NN = (((1,), (0,)), ((), ()))
_TN = (((0,), (0,)), ((), ()))
_TX = (((0,), (1,)), ((), ()))


def _moe_block(x_ref, We_ref, be_ref, Wg_ref, Wi_ref, bi_ref, Wo_ref, bo_ref,
               Wp_ref, bp_ref, tile16_ref, gsum_ref, s128_ref,
               prefix_ref, out_ref):
    f32 = jnp.float32
    dot = lambda a, b, dn=_NN: lax.dot_general(a, b, dn,
                                               preferred_element_type=f32)
    x = x_ref[...]
    h = dot(We_ref[...], x, _TX) + be_ref[...]
    logits = dot(Wg_ref[...], h, _TN)
    m = jnp.max(logits, axis=0, keepdims=True)
    el = jnp.exp(logits - m)
    gate = 1.0 / jnp.sum(el, axis=0, keepdims=True)
    is_max = (logits >= m).astype(f32)
    prior = dot(prefix_ref[...], is_max)
    onehot = jnp.where(prior == 0.0, is_max, 0.0)
    mask = dot(s128_ref[...], onehot)
    h_exp = dot(tile16_ref[...], h) * mask
    wi_rows = Wi_ref[...].reshape(E * D_HID, D_FF)
    pre = dot(wi_rows, h_exp, _TN) + dot(bi_ref[...], onehot, _TN)
    mid = jax.nn.gelu(pre)
    wo_all = jnp.transpose(Wo_ref[...], (0, 2, 1)).reshape(E * D_HID, D_FF)
    o_all = dot(wo_all, mid)
    moe = (dot(gsum_ref[...], o_all * mask)
           + dot(bo_ref[...], onehot, _TN)) * gate
    out_t = dot(Wp_ref[...], moe, _TN) + bp_ref[...]
    out_ref[...] = out_t.T


def kernel(x, W_embed, b_embed, W_gate, Wi, bi, Wo, bo, W_proj, b_proj):
    full = lambda a: pl.BlockSpec(a.shape, lambda i: (0,) * a.ndim)
    args = (W_embed, b_embed.reshape(D_HID, 1), W_gate, Wi, bi, Wo, bo,
            W_proj, b_proj.reshape(D_IN, 1),
            jnp.asarray(_TILE16_T), jnp.asarray(_GSUM),
            jnp.asarray(_S128_T), jnp.asarray(_PREFIX_T))
    return pl.pallas_call(
        _moe_block,
        grid=(T // TT,),
        in_specs=[pl.BlockSpec((TT, D_IN), lambda i: (i, 0))]
                 + [full(a) for a in args],
        out_specs=pl.BlockSpec((TT, D_IN), lambda i: (i, 0)),
        out_shape=jax.ShapeDtypeStruct((T, D_IN), jnp.float32),
        compiler_params=pltpu.CompilerParams(
            dimension_semantics=("parallel",)),
    )(x, *args)


# speedup vs baseline: 28.3702x; 1.0089x over previous
import jax
import jax.numpy as jnp
import numpy as np
from jax import lax
from jax.experimental import pallas as pl
from jax.experimental.pallas import tpu as pltpu

T = 32768
D_IN = 4
D_HID = 16
D_FF = 32
E = 8
TT = 8192

_EYE_E = np.eye(E, dtype=np.float32)
_TILE16_T = np.tile(np.eye(D_HID, dtype=np.float32), (E, 1))
_GSUM = np.tile(np.eye(D_HID, dtype=np.float32), (1, E))
_S128_T = np.repeat(_EYE_E, D_HID, axis=0)
_PREFIX_T = np.tril(np.ones((E, E), dtype=np.float32), -1)

_NN = (((1,), (0,)), ((), ()))
_TN = (((0,), (0,)), ((), ()))
_TX = (((0,), (1,)), ((), ()))


def _moe_block(x_ref, We_ref, be_ref, Wg_ref, Wi_ref, bi_ref, Wo_ref, bo_ref,
               Wp_ref, bp_ref, tile16_ref, gsum_ref, s128_ref,
               prefix_ref, out_ref):
    _moe_half(x_ref, We_ref, be_ref, Wg_ref, Wi_ref, bi_ref, Wo_ref, bo_ref,
              Wp_ref, bp_ref, tile16_ref, gsum_ref, s128_ref, prefix_ref,
              out_ref, 0)
    _moe_half(x_ref, We_ref, be_ref, Wg_ref, Wi_ref, bi_ref, Wo_ref, bo_ref,
              Wp_ref, bp_ref, tile16_ref, gsum_ref, s128_ref, prefix_ref,
              out_ref, 1)


def _moe_half(x_ref, We_ref, be_ref, Wg_ref, Wi_ref, bi_ref, Wo_ref, bo_ref,
              Wp_ref, bp_ref, tile16_ref, gsum_ref, s128_ref, prefix_ref,
              out_ref, half):
    f32 = jnp.float32
    dot = lambda a, b, dn=_NN: lax.dot_general(a, b, dn,
                                               preferred_element_type=f32)
    HH = TT // 2
    x = x_ref[pl.ds(half * HH, HH), :]
    h = dot(We_ref[...], x, _TX) + be_ref[...]
    logits = dot(Wg_ref[...], h, _TN)
    m = jnp.max(logits, axis=0, keepdims=True)
    el = jnp.exp(logits - m)
    gate = 1.0 / jnp.sum(el, axis=0, keepdims=True)
    bf16 = jnp.bfloat16
    is_max = (logits >= m).astype(bf16)
    prior = dot(prefix_ref[...], is_max)
    onehot = jnp.where(prior == 0.0, is_max, jnp.zeros_like(is_max))
    onehot_f = onehot.astype(f32)
    mask = dot(s128_ref[...], onehot)
    h_exp = dot(tile16_ref[...], h) * mask
    wi_rows = Wi_ref[...].reshape(E * D_HID, D_FF)
    pre = dot(wi_rows, h_exp, _TN) + dot(bi_ref[...], onehot_f, _TN)
    mid = jax.nn.gelu(pre)
    wo_all = jnp.transpose(Wo_ref[...], (0, 2, 1)).reshape(E * D_HID, D_FF)
    o_all = dot(wo_all, mid)
    moe = (dot(gsum_ref[...], o_all * mask)
           + dot(bo_ref[...], onehot_f, _TN)) * gate
    out_t = dot(Wp_ref[...], moe, _TN) + bp_ref[...]
    out_ref[pl.ds(half * HH, HH), :] = out_t.T


def kernel(x, W_embed, b_embed, W_gate, Wi, bi, Wo, bo, W_proj, b_proj):
    full = lambda a: pl.BlockSpec(a.shape, lambda i: (0,) * a.ndim)
    args = (W_embed, b_embed.reshape(D_HID, 1), W_gate, Wi, bi, Wo, bo,
            W_proj, b_proj.reshape(D_IN, 1),
            jnp.asarray(_TILE16_T), jnp.asarray(_GSUM),
            jnp.asarray(_S128_T, dtype=jnp.bfloat16),
            jnp.asarray(_PREFIX_T, dtype=jnp.bfloat16))
    return pl.pallas_call(
        _moe_block,
        grid=(T // TT,),
        in_specs=[pl.BlockSpec((TT, D_IN), lambda i: (i, 0))]
                 + [full(a) for a in args],
        out_specs=pl.BlockSpec((TT, D_IN), lambda i: (i, 0)),
        out_shape=jax.ShapeDtypeStruct((T, D_IN), jnp.float32),
        compiler_params=pltpu.CompilerParams(
            dimension_semantics=("parallel",)),
    )(x, *args)
